# baseline (speedup 1.0000x reference)
import functools

import jax
import jax.numpy as jnp
from jax import lax
from jax.experimental import pallas as pl
from jax.experimental.pallas import tpu as pltpu
from jax.experimental.pallas import tpu_sc as plsc

B = 4096
L = 50
D = 64
LANES = 16
NC, NS = 2, 16
NW = NC * NS
VPW = B // NW
CV = 2
ROWS = CV * L
NCH = VPW // CV
NSEG = D // LANES


_RU = 5


VOCAB = 100000
NBUF = 4


def _sc_pool_body(idx_hbm, table_wide_hbm, out_hbm, idx_v, rows_v, pooled_v,
                  sems):
    wid = lax.axis_index("s") * NC + lax.axis_index("c")
    pltpu.sync_copy(idx_hbm.at[wid], idx_v)

    def start(g, b):
        pltpu.make_async_copy(
            table_wide_hbm.at[idx_v.at[g]], rows_v.at[b], sems.at[b]).start()

    def wait(g, b):
        pltpu.make_async_copy(
            table_wide_hbm.at[idx_v.at[g]], rows_v.at[b], sems.at[b]).wait()

    def process(g, b):
        for v in range(CV):
            def row_body(r, accs):
                base = v * L + r * _RU
                for u in range(_RU):
                    accs = tuple(
                        accs[s] + rows_v[b, base + u, pl.ds(s * LANES, LANES)]
                        for s in range(NSEG)
                    )
                return accs
            zero = jnp.zeros((LANES,), jnp.float32)
            accs = lax.fori_loop(0, L // _RU, row_body, (zero,) * NSEG)
            vis = g * CV + v
            for s in range(NSEG):
                pooled_v[vis, pl.ds(s * LANES, LANES)] = accs[s]

    for b in range(NBUF):
        start(b, b)

    def loop_body(gq, carry):
        g = gq * NBUF
        for b in range(NBUF):
            wait(g + b, b)
            process(g + b, b)

            @pl.when(g + b + NBUF < NCH)
            def _():
                start(g + b + NBUF, b)
        return carry

    lax.fori_loop(0, NCH // NBUF, loop_body, 0)
    pltpu.sync_copy(pooled_v, out_hbm.at[pl.ds(wid * VPW, VPW)])


_sc_pool = pl.kernel(
    _sc_pool_body,
    out_type=jax.ShapeDtypeStruct((B, D), jnp.float32),
    mesh=plsc.VectorSubcoreMesh(core_axis_name="c", subcore_axis_name="s"),
    scratch_types=[
        pltpu.VMEM((NCH, ROWS), jnp.int32),
        pltpu.VMEM((NBUF, ROWS, 2 * D), jnp.float32),
        pltpu.VMEM((VPW, D), jnp.float32),
        pltpu.SemaphoreType.DMA((NBUF,)),
    ],
    compiler_params=pltpu.CompilerParams(use_tc_tiling_on_sc=False),
)


_PACK_BLK = 4000


def _tc_pack_body(in_ref, out_ref):
    x = in_ref[...]
    out_ref[...] = jnp.concatenate([x, x], axis=1)


def _tc_pack(table):
    return pl.pallas_call(
        _tc_pack_body,
        grid=(VOCAB // _PACK_BLK,),
        in_specs=[pl.BlockSpec((_PACK_BLK, D), lambda i: (i, 0))],
        out_specs=pl.BlockSpec((_PACK_BLK, 2 * D), lambda i: (i, 0)),
        out_shape=jax.ShapeDtypeStruct((VOCAB, 2 * D), jnp.float32),
    )(table)


def _tc_dense_body(pooled_ref, wt_ref, b_ref, g_ref, beta_ref, out_ref):
    h = jnp.maximum(pooled_ref[...] * (1.0 / L), 0.0)
    z = jnp.dot(h, wt_ref[...], preferred_element_type=jnp.float32) + b_ref[...]
    mu = jnp.mean(z, axis=-1, keepdims=True)
    zc = z - mu
    var = jnp.mean(zc * zc, axis=-1, keepdims=True)
    out_ref[...] = zc * lax.rsqrt(var + 1e-5) * g_ref[...] + beta_ref[...]


_TC_BLOCK = 512


@jax.jit
def _encode(visit_code_indices, emb_table, W_proj, b_proj, ln_gamma, ln_beta):
    idx = visit_code_indices.astype(jnp.int32).reshape(NW, NCH, ROWS)
    pooled = _sc_pool(idx, _tc_pack(emb_table))
    grid = B // _TC_BLOCK
    out = pl.pallas_call(
        _tc_dense_body,
        grid=(grid,),
        in_specs=[
            pl.BlockSpec((_TC_BLOCK, D), lambda i: (i, 0)),
            pl.BlockSpec((D, D), lambda i: (0, 0)),
            pl.BlockSpec((1, D), lambda i: (0, 0)),
            pl.BlockSpec((1, D), lambda i: (0, 0)),
            pl.BlockSpec((1, D), lambda i: (0, 0)),
        ],
        out_specs=pl.BlockSpec((_TC_BLOCK, D), lambda i: (i, 0)),
        out_shape=jax.ShapeDtypeStruct((B, D), jnp.float32),
    )(pooled, W_proj.T, b_proj[None, :], ln_gamma[None, :], ln_beta[None, :])
    return out


def kernel(visit_code_indices, emb_table, W_proj, b_proj, ln_gamma, ln_beta):
    return _encode(visit_code_indices, emb_table, W_proj, b_proj,
                   ln_gamma, ln_beta)


# speedup vs baseline: 1.3141x; 1.3141x over previous
import functools

import jax
import jax.numpy as jnp
from jax import lax
from jax.experimental import pallas as pl
from jax.experimental.pallas import tpu as pltpu
from jax.experimental.pallas import tpu_sc as plsc

B = 4096
L = 50
D = 64
LANES = 16
NC, NS = 2, 16
NW = NC * NS
VPW = B // NW
CV = 2
ROWS = CV * L
NCH = VPW // CV
NSEG = D // LANES


_RU = 5


VOCAB = 100000
NBUF = 4


def _sc_pool_body(idx_hbm, table_hbm, out_hbm, idx_v, rows_v, pooled_v,
                  sems):
    wid = lax.axis_index("s") * NC + lax.axis_index("c")
    pltpu.sync_copy(idx_hbm.at[wid], idx_v)

    def start(g, b):
        pltpu.make_async_copy(
            table_hbm.at[idx_v.at[g]], rows_v.at[b], sems.at[b]).start()

    def wait(g, b):
        pltpu.make_async_copy(
            table_hbm.at[idx_v.at[g]], rows_v.at[b], sems.at[b]).wait()

    def process(g, b):
        for v in range(CV):
            def row_body(r, accs):
                base = v * L + r * _RU
                for u in range(_RU):
                    accs = tuple(
                        accs[s] + rows_v[b, base + u, pl.ds(s * LANES, LANES)]
                        for s in range(NSEG)
                    )
                return accs
            zero = jnp.zeros((LANES,), jnp.float32)
            accs = lax.fori_loop(0, L // _RU, row_body, (zero,) * NSEG)
            vis = g * CV + v
            for s in range(NSEG):
                pooled_v[vis, pl.ds(s * LANES, LANES)] = accs[s]

    for b in range(NBUF):
        start(b, b)

    def loop_body(gq, carry):
        g = gq * NBUF
        for b in range(NBUF):
            wait(g + b, b)
            process(g + b, b)

            @pl.when(g + b + NBUF < NCH)
            def _():
                start(g + b + NBUF, b)
        return carry

    lax.fori_loop(0, NCH // NBUF, loop_body, 0)
    pltpu.sync_copy(pooled_v, out_hbm.at[pl.ds(wid * VPW, VPW)])


_sc_pool = pl.kernel(
    _sc_pool_body,
    out_type=jax.ShapeDtypeStruct((B, D), jnp.float32),
    mesh=plsc.VectorSubcoreMesh(core_axis_name="c", subcore_axis_name="s"),
    scratch_types=[
        pltpu.VMEM((NCH, ROWS), jnp.int32),
        pltpu.VMEM((NBUF, ROWS, D), jnp.float32),
        pltpu.VMEM((VPW, D), jnp.float32),
        pltpu.SemaphoreType.DMA((NBUF,)),
    ],
    compiler_params=pltpu.CompilerParams(use_tc_tiling_on_sc=False),
)


def _tc_dense_body(pooled_ref, wt_ref, b_ref, g_ref, beta_ref, out_ref):
    h = jnp.maximum(pooled_ref[...] * (1.0 / L), 0.0)
    z = jnp.dot(h, wt_ref[...], preferred_element_type=jnp.float32) + b_ref[...]
    mu = jnp.mean(z, axis=-1, keepdims=True)
    zc = z - mu
    var = jnp.mean(zc * zc, axis=-1, keepdims=True)
    out_ref[...] = zc * lax.rsqrt(var + 1e-5) * g_ref[...] + beta_ref[...]


_TC_BLOCK = 512


@jax.jit
def _encode(visit_code_indices, emb_table, W_proj, b_proj, ln_gamma, ln_beta):
    idx = visit_code_indices.astype(jnp.int32).reshape(NW, NCH, ROWS)
    pooled = _sc_pool(idx, emb_table)
    grid = B // _TC_BLOCK
    out = pl.pallas_call(
        _tc_dense_body,
        grid=(grid,),
        in_specs=[
            pl.BlockSpec((_TC_BLOCK, D), lambda i: (i, 0)),
            pl.BlockSpec((D, D), lambda i: (0, 0)),
            pl.BlockSpec((1, D), lambda i: (0, 0)),
            pl.BlockSpec((1, D), lambda i: (0, 0)),
            pl.BlockSpec((1, D), lambda i: (0, 0)),
        ],
        out_specs=pl.BlockSpec((_TC_BLOCK, D), lambda i: (i, 0)),
        out_shape=jax.ShapeDtypeStruct((B, D), jnp.float32),
    )(pooled, W_proj.T, b_proj[None, :], ln_gamma[None, :], ln_beta[None, :])
    return out


def kernel(visit_code_indices, emb_table, W_proj, b_proj, ln_gamma, ln_beta):
    return _encode(visit_code_indices, emb_table, W_proj, b_proj,
                   ln_gamma, ln_beta)


# speedup vs baseline: 1.3403x; 1.0200x over previous
import functools

import jax
import jax.numpy as jnp
from jax import lax
from jax.experimental import pallas as pl
from jax.experimental.pallas import tpu as pltpu
from jax.experimental.pallas import tpu_sc as plsc

B = 4096
L = 50
D = 64
LANES = 16
NC, NS = 2, 16
NW = NC * NS
VPW = B // NW
CV = 2
ROWS = CV * L
NCH = VPW // CV
NSEG = D // LANES


_RU = 10


VOCAB = 100000
NBUF = 8


def _sc_pool_body(idx_hbm, table_hbm, out_hbm, idx_v, rows_v, pooled_v,
                  sems):
    wid = lax.axis_index("s") * NC + lax.axis_index("c")
    pltpu.sync_copy(idx_hbm.at[wid], idx_v)

    def start(g, b):
        pltpu.make_async_copy(
            table_hbm.at[idx_v.at[g]], rows_v.at[b], sems.at[b]).start()

    def wait(g, b):
        pltpu.make_async_copy(
            table_hbm.at[idx_v.at[g]], rows_v.at[b], sems.at[b]).wait()

    def process(g, b):
        for v in range(CV):
            def row_body(r, accs):
                base = v * L + r * _RU
                for u in range(_RU):
                    accs = tuple(
                        accs[s] + rows_v[b, base + u, pl.ds(s * LANES, LANES)]
                        for s in range(NSEG)
                    )
                return accs
            zero = jnp.zeros((LANES,), jnp.float32)
            accs = lax.fori_loop(0, L // _RU, row_body, (zero,) * NSEG)
            vis = g * CV + v
            for s in range(NSEG):
                pooled_v[vis, pl.ds(s * LANES, LANES)] = accs[s]

    for b in range(NBUF):
        start(b, b)

    def loop_body(gq, carry):
        g = gq * NBUF
        for b in range(NBUF):
            wait(g + b, b)
            process(g + b, b)

            @pl.when(g + b + NBUF < NCH)
            def _():
                start(g + b + NBUF, b)
        return carry

    lax.fori_loop(0, NCH // NBUF, loop_body, 0)
    pltpu.sync_copy(pooled_v, out_hbm.at[pl.ds(wid * VPW, VPW)])


_sc_pool = pl.kernel(
    _sc_pool_body,
    out_type=jax.ShapeDtypeStruct((B, D), jnp.float32),
    mesh=plsc.VectorSubcoreMesh(core_axis_name="c", subcore_axis_name="s"),
    scratch_types=[
        pltpu.VMEM((NCH, ROWS), jnp.int32),
        pltpu.VMEM((NBUF, ROWS, D), jnp.float32),
        pltpu.VMEM((VPW, D), jnp.float32),
        pltpu.SemaphoreType.DMA((NBUF,)),
    ],
    compiler_params=pltpu.CompilerParams(use_tc_tiling_on_sc=False),
)


def _tc_dense_body(pooled_ref, wt_ref, b_ref, g_ref, beta_ref, out_ref):
    h = jnp.maximum(pooled_ref[...] * (1.0 / L), 0.0)
    z = jnp.dot(h, wt_ref[...], preferred_element_type=jnp.float32) + b_ref[...]
    mu = jnp.mean(z, axis=-1, keepdims=True)
    zc = z - mu
    var = jnp.mean(zc * zc, axis=-1, keepdims=True)
    out_ref[...] = zc * lax.rsqrt(var + 1e-5) * g_ref[...] + beta_ref[...]


_TC_BLOCK = 512


@jax.jit
def _encode(visit_code_indices, emb_table, W_proj, b_proj, ln_gamma, ln_beta):
    idx = visit_code_indices.astype(jnp.int32).reshape(NW, NCH, ROWS)
    pooled = _sc_pool(idx, emb_table)
    grid = B // _TC_BLOCK
    out = pl.pallas_call(
        _tc_dense_body,
        grid=(grid,),
        in_specs=[
            pl.BlockSpec((_TC_BLOCK, D), lambda i: (i, 0)),
            pl.BlockSpec((D, D), lambda i: (0, 0)),
            pl.BlockSpec((1, D), lambda i: (0, 0)),
            pl.BlockSpec((1, D), lambda i: (0, 0)),
            pl.BlockSpec((1, D), lambda i: (0, 0)),
        ],
        out_specs=pl.BlockSpec((_TC_BLOCK, D), lambda i: (i, 0)),
        out_shape=jax.ShapeDtypeStruct((B, D), jnp.float32),
    )(pooled, W_proj.T, b_proj[None, :], ln_gamma[None, :], ln_beta[None, :])
    return out


def kernel(visit_code_indices, emb_table, W_proj, b_proj, ln_gamma, ln_beta):
    return _encode(visit_code_indices, emb_table, W_proj, b_proj,
                   ln_gamma, ln_beta)
